# Initial kernel scaffold; baseline (speedup 1.0000x reference)
import jax
import jax.numpy as jnp
from jax.experimental import pallas as pl


def kernel(token_ids, embedding, w_ih, w_hh, b, w_fc, b_fc):
    raise NotImplementedError("write your pallas kernel here")


# speedup vs baseline: 2.8749x; 2.8749x over previous
import functools

import jax
import jax.numpy as jnp
from jax.experimental import pallas as pl
from jax.experimental.pallas import tpu as pltpu


def _round_up(x, m):
    return -(-x // m) * m


def _sigmoid(x):
    return 0.5 * (jnp.tanh(0.5 * x) + 1.0)


def _lstm_kernel(ids_ref, emb_ref, wih_ref, whh_ref, b_ref, wfc_ref, bfc_ref,
                 out_ref, *, seq_len, bb):
    E = emb_ref.shape[1]
    H = whh_ref.shape[0]
    runs = bb // 8
    base = pl.program_id(0) * ((seq_len + 1) * bb)
    row_iota = jax.lax.broadcasted_iota(jnp.int32, (8, E), 0)

    def gather_step(tbase):
        pieces = []
        for r in range(runs):
            acc = jnp.zeros((8, E), jnp.float32)
            for k in range(8):
                mi = r * 8 + k
                v = ids_ref[tbase + mi]
                c8 = pl.multiple_of((v >> 3) << 3, 8)
                chunk = emb_ref[pl.ds(c8, 8), :]
                rolled = pltpu.roll(chunk, (mi - v) & 7, axis=0)
                acc = jnp.where(row_iota == k, rolled, acc)
            pieces.append(acc)
        return jnp.concatenate(pieces, axis=0)

    def step(t, carry):
        h, c, x = carry
        gates = (jnp.dot(x, wih_ref[...], preferred_element_type=jnp.float32)
                 + jnp.dot(h.astype(jnp.bfloat16), whh_ref[...],
                           preferred_element_type=jnp.float32)
                 + b_ref[...])
        i_g = _sigmoid(gates[:, 0 * H:1 * H])
        f_g = _sigmoid(gates[:, 1 * H:2 * H])
        g_g = jnp.tanh(gates[:, 2 * H:3 * H])
        o_g = _sigmoid(gates[:, 3 * H:4 * H])
        c_new = f_g * c + i_g * g_g
        h_new = o_g * jnp.tanh(c_new)
        x_next = gather_step(base + (t + 1) * bb).astype(jnp.bfloat16)
        return h_new, c_new, x_next

    h0 = jnp.zeros((bb, H), jnp.float32)
    c0 = jnp.zeros((bb, H), jnp.float32)
    x0 = gather_step(base).astype(jnp.bfloat16)
    h, c, _ = jax.lax.fori_loop(0, seq_len, step, (h0, c0, x0))

    logits = (jnp.dot(h.astype(jnp.bfloat16), wfc_ref[...],
                      preferred_element_type=jnp.float32) + bfc_ref[...])
    out_ref[...] = _sigmoid(logits)


def kernel(token_ids, embedding, w_ih, w_hh, b, w_fc, b_fc):
    B, T = token_ids.shape
    V, E = embedding.shape
    H = w_hh.shape[0]
    O = w_fc.shape[1]

    n_cores = 2
    bb = B // n_cores

    ids = token_ids.astype(jnp.int32).reshape(n_cores, bb, T).transpose(0, 2, 1)
    ids = jnp.pad(ids, ((0, 0), (0, 1), (0, 0))).reshape(-1)

    emb = embedding
    if V % 8:
        emb = jnp.pad(emb, ((0, _round_up(V, 8) - V), (0, 0)))

    wih = w_ih.astype(jnp.bfloat16)
    whh = w_hh.astype(jnp.bfloat16)
    O_pad = max(128, _round_up(O, 128))
    wfc = jnp.pad(w_fc, ((0, 0), (0, O_pad - O))).astype(jnp.bfloat16)
    bfc = jnp.pad(b_fc, ((0, 0), (0, O_pad - O)))

    kfn = functools.partial(_lstm_kernel, seq_len=T, bb=bb)

    out = pl.pallas_call(
        kfn,
        out_shape=jax.ShapeDtypeStruct((B, O_pad), jnp.float32),
        grid_spec=pltpu.PrefetchScalarGridSpec(
            num_scalar_prefetch=1,
            grid=(n_cores,),
            in_specs=[
                pl.BlockSpec(emb.shape, lambda i, ids: (0, 0)),
                pl.BlockSpec(wih.shape, lambda i, ids: (0, 0)),
                pl.BlockSpec(whh.shape, lambda i, ids: (0, 0)),
                pl.BlockSpec(b.shape, lambda i, ids: (0, 0)),
                pl.BlockSpec(wfc.shape, lambda i, ids: (0, 0)),
                pl.BlockSpec(bfc.shape, lambda i, ids: (0, 0)),
            ],
            out_specs=pl.BlockSpec((bb, O_pad), lambda i, ids: (i, 0)),
        ),
        compiler_params=pltpu.CompilerParams(
            dimension_semantics=("parallel",),
            vmem_limit_bytes=48 << 20),
    )(ids, emb, wih, whh, b, wfc, bfc)

    return out[:, :O]
